# Initial kernel scaffold; baseline (speedup 1.0000x reference)
import jax
import jax.numpy as jnp
from jax.experimental import pallas as pl


def kernel(users, items, edge_row, edge_col, edge_vals, user_table, item_table):
    raise NotImplementedError("write your pallas kernel here")


# speedup vs baseline: 2.4128x; 2.4128x over previous
import functools

import jax
import jax.numpy as jnp
from jax import lax
from jax.experimental import pallas as pl
from jax.experimental.pallas import tpu as pltpu
from jax.experimental.pallas import tpu_sc as plsc

NUM_USERS = 5000
NUM_ITEMS = 5000
D = 128
N_NODES = NUM_USERS + NUM_ITEMS
N_EDGES = 320000
N_LAYERS = 3
BATCH = 4096

NC = 2
NS = 16
NW = NC * NS
L = 16

CH = 80
G = 32
NG = 4
NCHUNK = G * NG
EPT = NCHUNK * CH
NEP = EPT * NW
NP = 10240
RPT = NP // NS
RCH = 64
NRCH = RPT // RCH
PPT = BATCH // NW

_f32 = jnp.float32
_i32 = jnp.int32


def _mesh():
  return plsc.VectorSubcoreMesh(core_axis_name="c", subcore_axis_name="s",
                                num_cores=NC, num_subcores=NS)


def _layer_body(e_ref, col_ref, row_ref, val_ref, part_ref,
                colv, rowv, valv, rowsv, accv, acc, sem):
  cid = lax.axis_index("c")
  tid = lax.axis_index("s")

  zv = jnp.zeros((L,), _f32)

  def zero_row(i, _):
    for q in range(D // L):
      accv[i, pl.ds(q * L, L)] = zv
    return 0

  lax.fori_loop(0, RCH, zero_row, 0)

  def zero_copy(r, _):
    pltpu.sync_copy(accv, acc.at[pl.ds(tid * RPT + r * RCH, RCH)])
    return 0

  lax.fori_loop(0, NRCH, zero_copy, 0)
  plsc.subcore_barrier()

  def group_body(gi, _):
    pltpu.sync_copy(col_ref.at[cid, tid, pl.ds(gi * G, G)], colv)
    pltpu.sync_copy(row_ref.at[cid, tid, pl.ds(gi * G, G)], rowv)
    pltpu.sync_copy(val_ref.at[cid, tid, pl.ds(gi * G, G)], valv)

    def chunk_body(j, _):
      pltpu.async_copy(e_ref.at[colv.at[j]], rowsv, sem).wait()

      def vgrp_body(g, _):
        v16 = valv[j, pl.ds(g * L, L)]
        base = g * L
        for r in range(L):
          v = v16[r]
          for q in range(D // L):
            s = pl.ds(q * L, L)
            rowsv[base + r, s] = rowsv[base + r, s] * v
        return 0

      lax.fori_loop(0, CH // L, vgrp_body, 0)
      pltpu.sync_copy(rowsv, acc.at[rowv.at[j]], add=True)
      return 0

    lax.fori_loop(0, G, chunk_body, 0)
    return 0

  lax.fori_loop(0, NG, group_body, 0)
  plsc.subcore_barrier()

  def drain(r, _):
    r0 = tid * RPT + r * RCH
    pltpu.sync_copy(acc.at[pl.ds(r0, RCH)], accv)
    pltpu.sync_copy(accv, part_ref.at[cid, pl.ds(r0, RCH)])
    return 0

  lax.fori_loop(0, NRCH, drain, 0)


def _layer_call(e_in, colb, rowb, valb):
  k = functools.partial(
      pl.kernel,
      out_type=jax.ShapeDtypeStruct((NC, NP, D), _f32),
      mesh=_mesh(),
      scratch_types=[
          pltpu.VMEM((G, CH), _i32),
          pltpu.VMEM((G, CH), _i32),
          pltpu.VMEM((G, CH), _f32),
          pltpu.VMEM((CH, D), _f32),
          pltpu.VMEM((RCH, D), _f32),
          pltpu.VMEM_SHARED((NP, D), _f32),
          pltpu.SemaphoreType.DMA,
      ],
  )(_layer_body)
  return k(e_in, colb, rowb, valb)


def _norm_body(x_ref, o_ref):
  x = x_ref[...]
  n = jnp.sqrt(jnp.sum(x * x, axis=1, keepdims=True))
  o_ref[...] = x / jnp.maximum(n, 1e-12)


def _norm_call(x):
  blk = 1024
  return pl.pallas_call(
      _norm_body,
      out_shape=jax.ShapeDtypeStruct((NP, D), _f32),
      grid=(NP // blk,),
      in_specs=[pl.BlockSpec((blk, D), lambda j: (j, 0))],
      out_specs=pl.BlockSpec((blk, D), lambda j: (j, 0)),
  )(x)


def _combine_body(p_ref, s_ref, e_ref, so_ref):
  e = p_ref[0] + p_ref[1]
  e_ref[...] = e
  so_ref[...] = s_ref[...] + e


def _combine_call(parts, sum_in):
  blk = 1024
  return pl.pallas_call(
      _combine_body,
      out_shape=(jax.ShapeDtypeStruct((NP, D), _f32),
                 jax.ShapeDtypeStruct((NP, D), _f32)),
      grid=(NP // blk,),
      in_specs=[pl.BlockSpec((NC, blk, D), lambda j: (0, j, 0)),
                pl.BlockSpec((blk, D), lambda j: (j, 0))],
      out_specs=(pl.BlockSpec((blk, D), lambda j: (j, 0)),
                 pl.BlockSpec((blk, D), lambda j: (j, 0))),
  )(parts, sum_in)


def _gather_body(s_ref, u_ref, i_ref, ug_ref, ig_ref,
                 uidx, iidx, urows, irows, sem):
  cid = lax.axis_index("c")
  tid = lax.axis_index("s")
  pltpu.sync_copy(u_ref.at[cid, tid], uidx)
  pltpu.sync_copy(i_ref.at[cid, tid], iidx)
  for q in range(PPT // L):
    s = pl.ds(q * L, L)
    iidx[s] = iidx[s] + NUM_USERS
  pltpu.async_copy(s_ref.at[uidx], urows, sem).wait()
  pltpu.async_copy(s_ref.at[iidx], irows, sem).wait()
  wid = cid * NS + tid
  pltpu.sync_copy(urows, ug_ref.at[pl.ds(wid * PPT, PPT)])
  pltpu.sync_copy(irows, ig_ref.at[pl.ds(wid * PPT, PPT)])


def _gather_call(sum_emb, users, items):
  k = functools.partial(
      pl.kernel,
      out_type=(jax.ShapeDtypeStruct((BATCH, D), _f32),
                jax.ShapeDtypeStruct((BATCH, D), _f32)),
      mesh=_mesh(),
      scratch_types=[
          pltpu.VMEM((PPT,), _i32),
          pltpu.VMEM((PPT,), _i32),
          pltpu.VMEM((PPT, D), _f32),
          pltpu.VMEM((PPT, D), _f32),
          pltpu.SemaphoreType.DMA,
      ],
  )(_gather_body)
  return k(sum_emb, users, items)


def _dot_body(u_ref, i_ref, o_ref):
  d = jnp.sum(u_ref[...] * i_ref[...], axis=1) * (1.0 / 16.0)
  o_ref[...] = d.reshape(o_ref.shape)


def _dot_call(ug, ig):
  g = pl.pallas_call(
      _dot_body,
      out_shape=jax.ShapeDtypeStruct((8, BATCH // 8), _f32),
  )(ug, ig)
  return g.reshape(BATCH)


def kernel(users, items, edge_row, edge_col, edge_vals, user_table, item_table):
  npad = NEP - N_EDGES
  col = jnp.concatenate([edge_col.astype(_i32), jnp.zeros((npad,), _i32)])
  row = jnp.concatenate(
      [edge_row.astype(_i32), jnp.full((npad,), N_NODES, _i32)])
  val = jnp.concatenate([edge_vals.astype(_f32), jnp.zeros((npad,), _f32)])
  colb = col.reshape(NC, NS, NCHUNK, CH)
  rowb = row.reshape(NC, NS, NCHUNK, CH)
  valb = val.reshape(NC, NS, NCHUNK, CH)
  ub = users.astype(_i32).reshape(NC, NS, PPT)
  ib = items.astype(_i32).reshape(NC, NS, PPT)

  emb = jnp.concatenate([user_table, item_table], axis=0)
  emb = jnp.pad(emb, ((0, NP - N_NODES), (0, 0)), constant_values=1.0)
  e0 = _norm_call(emb)
  e = e0
  s = e0
  for _ in range(N_LAYERS):
    parts = _layer_call(e, colb, rowb, valb)
    e, s = _combine_call(parts, s)
  ug, ig = _gather_call(s, ub, ib)
  return _dot_call(ug, ig)
